# baseline (speedup 1.0000x reference)
import functools

import jax
import jax.numpy as jnp
from jax import lax
from jax.experimental import pallas as pl
from jax.experimental.pallas import tpu as pltpu
from jax.experimental.pallas import tpu_sc as plsc

N = 10000
E = 640000
IN_DIM = 16
HID = 128
OUT_DIM = 10000

NP = 10112
PAD_ROW = N
NW = 32
K = 128
CPW = 160
PW = CPW * K
EPAD = NW * PW
IB = 8
RPT = NP // 16

_mesh = plsc.VectorSubcoreMesh(core_axis_name="c", subcore_axis_name="s")


@functools.partial(
    pl.kernel,
    out_type=jax.ShapeDtypeStruct((NW, NP), jnp.float32),
    mesh=_mesh,
    scratch_types=[
        pltpu.VMEM((K,), jnp.int32),
        pltpu.VMEM((NP,), jnp.float32),
    ],
    compiler_params=pltpu.CompilerParams(needs_layout_passes=False),
)
def _deg_kernel(dst_hbm, out_hbm, idx_d, acc):
    c = lax.axis_index("c")
    s = lax.axis_index("s")
    wid = c * 16 + s
    zeros16 = jnp.zeros((16,), jnp.float32)
    ones16 = jnp.ones((16,), jnp.float32)

    def zbody(i, carry):
        acc[pl.ds(i * 16, 16)] = zeros16
        return carry

    lax.fori_loop(0, NP // 16, zbody, 0)

    base0 = pl.multiple_of(wid * PW, 8)

    def body(i, carry):
        base = pl.multiple_of(base0 + i * K, 8)
        pltpu.sync_copy(dst_hbm.at[pl.ds(base, K)], idx_d)
        for j in range(K // 16):
            idx16 = idx_d[pl.ds(j * 16, 16)]
            plsc.addupdate_scatter(acc, [idx16], ones16)
        return carry

    lax.fori_loop(0, CPW, body, 0)
    pltpu.sync_copy(acc, out_hbm.at[wid])


@functools.partial(
    pl.kernel,
    out_type=jax.ShapeDtypeStruct((2, NP, HID), jnp.float32),
    mesh=_mesh,
    scratch_types=[
        pltpu.VMEM((K,), jnp.int32),
        pltpu.VMEM((K,), jnp.int32),
        pltpu.VMEM((K, HID), jnp.float32),
        pltpu.VMEM_SHARED((NP, HID), jnp.float32),
        pltpu.SemaphoreType.DMA,
    ],
)
def _agg_kernel(y_hbm, src_hbm, dst_hbm, zero_hbm, out_hbm,
                idx_s, idx_d, rows, acc, sem):
    c = lax.axis_index("c")
    s = lax.axis_index("s")
    wid = c * 16 + s

    row0 = pl.multiple_of(s * RPT, 8)
    pltpu.sync_copy(zero_hbm, acc.at[pl.ds(row0, RPT)])
    plsc.subcore_barrier()

    base0 = pl.multiple_of(wid * PW, 8)

    def body(i, carry):
        base = pl.multiple_of(base0 + i * K, 8)
        pltpu.sync_copy(src_hbm.at[pl.ds(base, K)], idx_s)
        pltpu.sync_copy(dst_hbm.at[pl.ds(base, K)], idx_d)
        pltpu.async_copy(y_hbm.at[idx_s], rows, sem).wait()
        pltpu.sync_copy(rows, acc.at[idx_d], add=True)
        return carry

    lax.fori_loop(0, CPW, body, 0)
    plsc.subcore_barrier()
    pltpu.sync_copy(acc.at[pl.ds(row0, RPT)],
                    out_hbm.at[c, pl.ds(row0, RPT)])


def _prep_body(degp_ref, x_ref, w1_ref, dis_ref, y1_ref):
    deg = jnp.sum(degp_ref[...], axis=1, keepdims=True) + 1.0
    dis = lax.rsqrt(deg)
    dis_ref[...] = dis
    xw = jnp.dot(x_ref[...], w1_ref[...], preferred_element_type=jnp.float32)
    y1_ref[...] = dis * xw


def _prep(degp_t, xpad, w1):
    return pl.pallas_call(
        _prep_body,
        out_shape=(
            jax.ShapeDtypeStruct((NP, 1), jnp.float32),
            jax.ShapeDtypeStruct((NP, HID), jnp.float32),
        ),
    )(degp_t, xpad, w1)


_MR = 2528


def _mid_body(aggp_ref, y_ref, dis_ref, b_ref, w_ref, out_ref):
    dis = dis_ref[...]
    tot = aggp_ref[0] + aggp_ref[1] + y_ref[...]
    h = jnp.maximum(dis * tot + b_ref[...], 0.0)
    out_ref[...] = dis * jnp.dot(h, w_ref[...],
                                 preferred_element_type=jnp.float32)


def _mid(aggp, y, dis, b, w):
    return pl.pallas_call(
        _mid_body,
        grid=(NP // _MR,),
        in_specs=[
            pl.BlockSpec((2, _MR, HID), lambda i: (0, i, 0)),
            pl.BlockSpec((_MR, HID), lambda i: (i, 0)),
            pl.BlockSpec((_MR, 1), lambda i: (i, 0)),
            pl.BlockSpec((1, HID), lambda i: (0, 0)),
            pl.BlockSpec((HID, HID), lambda i: (0, 0)),
        ],
        out_specs=pl.BlockSpec((_MR, HID), lambda i: (i, 0)),
        out_shape=jax.ShapeDtypeStruct((NP, HID), jnp.float32),
    )(aggp, y, dis, b, w)


def _combine_body(aggp_ref, y_ref, dis_ref, b_ref, out_ref):
    dis = dis_ref[...]
    tot = aggp_ref[0] + aggp_ref[1] + y_ref[...]
    out_ref[...] = jnp.maximum(dis * tot + b_ref[...], 0.0)


def _combine(aggp, y, dis, b):
    return pl.pallas_call(
        _combine_body,
        grid=(NP // _MR,),
        in_specs=[
            pl.BlockSpec((2, _MR, HID), lambda i: (0, i, 0)),
            pl.BlockSpec((_MR, HID), lambda i: (i, 0)),
            pl.BlockSpec((_MR, 1), lambda i: (i, 0)),
            pl.BlockSpec((1, HID), lambda i: (0, 0)),
        ],
        out_specs=pl.BlockSpec((_MR, HID), lambda i: (i, 0)),
        out_shape=jax.ShapeDtypeStruct((NP, HID), jnp.float32),
    )(aggp, y, dis, b)


_FR = 2000
_FC = 2048


def _fc_body(h_ref, w_ref, b_ref, out_ref):
    out_ref[...] = jnp.dot(h_ref[...], w_ref[...],
                           preferred_element_type=jnp.float32) + b_ref[...]


def _fc(h, wfc, bfc):
    return pl.pallas_call(
        _fc_body,
        grid=(pl.cdiv(OUT_DIM, _FC), N // _FR),
        in_specs=[
            pl.BlockSpec((_FR, HID), lambda j, i: (i, 0)),
            pl.BlockSpec((HID, _FC), lambda j, i: (0, j)),
            pl.BlockSpec((1, _FC), lambda j, i: (0, j)),
        ],
        out_specs=pl.BlockSpec((_FR, _FC), lambda j, i: (i, j)),
        out_shape=jax.ShapeDtypeStruct((N, OUT_DIM), jnp.float32),
    )(h, wfc, bfc)


def kernel(x, edge_index, W1, b1, W2, b2, W3, b3, Wfc, bfc):
    src = edge_index[0].astype(jnp.int32)
    dst = edge_index[1].astype(jnp.int32)
    pad = jnp.full((EPAD - E,), PAD_ROW, jnp.int32)
    srcp = jnp.concatenate([src, pad])
    dstp = jnp.concatenate([dst, pad])
    xpad = jnp.pad(x, ((0, NP - N), (0, 0)))
    zero_rows = jnp.zeros((RPT, HID), jnp.float32)

    degp = _deg_kernel(dstp)
    degp_t = degp.T
    dis, y1 = _prep(degp_t, xpad, W1)

    p1 = _agg_kernel(y1, srcp, dstp, zero_rows)
    y2 = _mid(p1, y1, dis, b1.reshape(1, HID), W2)
    p2 = _agg_kernel(y2, srcp, dstp, zero_rows)
    y3 = _mid(p2, y2, dis, b2.reshape(1, HID), W3)
    p3 = _agg_kernel(y3, srcp, dstp, zero_rows)
    h3 = _combine(p3, y3, dis, b3.reshape(1, HID))

    return _fc(h3[:N], Wfc, bfc.reshape(1, OUT_DIM))


# speedup vs baseline: 2.1825x; 2.1825x over previous
import functools

import jax
import jax.numpy as jnp
from jax import lax
from jax.experimental import pallas as pl
from jax.experimental.pallas import tpu as pltpu
from jax.experimental.pallas import tpu_sc as plsc

N = 10000
E = 640000
IN_DIM = 16
HID = 128
OUT_DIM = 10000

NP = 10112
PAD_ROW = N
NW = 32
K = 128
CPW = 160
PW = CPW * K
EPAD = NW * PW
IB = 8
RPT = NP // 16

_mesh = plsc.VectorSubcoreMesh(core_axis_name="c", subcore_axis_name="s")


@functools.partial(
    pl.kernel,
    out_type=jax.ShapeDtypeStruct((NW, NP), jnp.float32),
    mesh=_mesh,
    scratch_types=[
        pltpu.VMEM((K,), jnp.int32),
        pltpu.VMEM((NP,), jnp.float32),
    ],
    compiler_params=pltpu.CompilerParams(needs_layout_passes=False),
)
def _deg_kernel(dst_hbm, out_hbm, idx_d, acc):
    c = lax.axis_index("c")
    s = lax.axis_index("s")
    wid = c * 16 + s
    zeros16 = jnp.zeros((16,), jnp.float32)
    ones16 = jnp.ones((16,), jnp.float32)

    def zbody(i, carry):
        acc[pl.ds(i * 16, 16)] = zeros16
        return carry

    lax.fori_loop(0, NP // 16, zbody, 0)

    base0 = pl.multiple_of(wid * PW, 8)

    def body(i, carry):
        base = pl.multiple_of(base0 + i * K, 8)
        pltpu.sync_copy(dst_hbm.at[pl.ds(base, K)], idx_d)
        for j in range(K // 16):
            idx16 = idx_d[pl.ds(j * 16, 16)]
            plsc.addupdate_scatter(acc, [idx16], ones16)
        return carry

    lax.fori_loop(0, CPW, body, 0)
    pltpu.sync_copy(acc, out_hbm.at[wid])


@functools.partial(
    pl.kernel,
    out_type=jax.ShapeDtypeStruct((2, NP, HID), jnp.float32),
    mesh=_mesh,
    scratch_types=[
        pltpu.VMEM((K,), jnp.int32),
        pltpu.VMEM((K,), jnp.int32),
        pltpu.VMEM((K, HID), jnp.float32),
        pltpu.VMEM_SHARED((NP, HID), jnp.float32),
        pltpu.SemaphoreType.DMA,
    ],
)
def _agg_kernel(y_hbm, src_hbm, dst_hbm, zero_hbm, out_hbm,
                idx_s, idx_d, rows, acc, sem):
    c = lax.axis_index("c")
    s = lax.axis_index("s")
    wid = c * 16 + s

    row0 = pl.multiple_of(s * RPT, 8)
    pltpu.sync_copy(zero_hbm, acc.at[pl.ds(row0, RPT)])
    plsc.subcore_barrier()

    base0 = pl.multiple_of(wid * PW, 8)

    def body(i, carry):
        base = pl.multiple_of(base0 + i * K, 8)
        pltpu.sync_copy(src_hbm.at[pl.ds(base, K)], idx_s)
        pltpu.sync_copy(dst_hbm.at[pl.ds(base, K)], idx_d)
        pltpu.async_copy(y_hbm.at[idx_s], rows, sem).wait()
        pltpu.sync_copy(rows, acc.at[idx_d], add=True)
        return carry

    lax.fori_loop(0, CPW, body, 0)
    plsc.subcore_barrier()
    pltpu.sync_copy(acc.at[pl.ds(row0, RPT)],
                    out_hbm.at[c, pl.ds(row0, RPT)])


def _prep_body(degp_ref, x_ref, w1_ref, dis_ref, y1_ref):
    deg = jnp.sum(degp_ref[...], axis=1, keepdims=True) + 1.0
    dis = lax.rsqrt(deg)
    dis_ref[...] = dis
    xw = jnp.dot(x_ref[...], w1_ref[...], preferred_element_type=jnp.float32)
    y1_ref[...] = dis * xw


def _prep(degp_t, xpad, w1):
    return pl.pallas_call(
        _prep_body,
        out_shape=(
            jax.ShapeDtypeStruct((NP, 1), jnp.float32),
            jax.ShapeDtypeStruct((NP, HID), jnp.float32),
        ),
    )(degp_t, xpad, w1)


_MR = 2528


def _mid_body(aggp_ref, y_ref, dis_ref, b_ref, w_ref, out_ref):
    dis = dis_ref[...]
    tot = aggp_ref[0] + aggp_ref[1] + y_ref[...]
    h = jnp.maximum(dis * tot + b_ref[...], 0.0)
    out_ref[...] = dis * jnp.dot(h, w_ref[...],
                                 preferred_element_type=jnp.float32)


def _mid(aggp, y, dis, b, w):
    return pl.pallas_call(
        _mid_body,
        grid=(NP // _MR,),
        in_specs=[
            pl.BlockSpec((2, _MR, HID), lambda i: (0, i, 0)),
            pl.BlockSpec((_MR, HID), lambda i: (i, 0)),
            pl.BlockSpec((_MR, 1), lambda i: (i, 0)),
            pl.BlockSpec((1, HID), lambda i: (0, 0)),
            pl.BlockSpec((HID, HID), lambda i: (0, 0)),
        ],
        out_specs=pl.BlockSpec((_MR, HID), lambda i: (i, 0)),
        out_shape=jax.ShapeDtypeStruct((NP, HID), jnp.float32),
    )(aggp, y, dis, b, w)


def _combine_body(aggp_ref, y_ref, dis_ref, b_ref, out_ref):
    dis = dis_ref[...]
    tot = aggp_ref[0] + aggp_ref[1] + y_ref[...]
    out_ref[...] = jnp.maximum(dis * tot + b_ref[...], 0.0)


def _combine(aggp, y, dis, b):
    return pl.pallas_call(
        _combine_body,
        grid=(NP // _MR,),
        in_specs=[
            pl.BlockSpec((2, _MR, HID), lambda i: (0, i, 0)),
            pl.BlockSpec((_MR, HID), lambda i: (i, 0)),
            pl.BlockSpec((_MR, 1), lambda i: (i, 0)),
            pl.BlockSpec((1, HID), lambda i: (0, 0)),
        ],
        out_specs=pl.BlockSpec((_MR, HID), lambda i: (i, 0)),
        out_shape=jax.ShapeDtypeStruct((NP, HID), jnp.float32),
    )(aggp, y, dis, b)


_FR = 2000
_FC = 2048


def _fc_body(h_ref, w_ref, b_ref, out_ref):
    out_ref[...] = jnp.dot(h_ref[...], w_ref[...],
                           preferred_element_type=jnp.float32) + b_ref[...]


def _fc(h, wfc, bfc):
    return pl.pallas_call(
        _fc_body,
        grid=(pl.cdiv(OUT_DIM, _FC), N // _FR),
        in_specs=[
            pl.BlockSpec((_FR, HID), lambda j, i: (i, 0)),
            pl.BlockSpec((HID, _FC), lambda j, i: (0, j)),
            pl.BlockSpec((1, _FC), lambda j, i: (0, j)),
        ],
        out_specs=pl.BlockSpec((_FR, _FC), lambda j, i: (i, j)),
        out_shape=jax.ShapeDtypeStruct((N, OUT_DIM), jnp.float32),
    )(h, wfc, bfc)


def kernel(x, edge_index, W1, b1, W2, b2, W3, b3, Wfc, bfc):
    src = edge_index[0].astype(jnp.int32)
    dst = edge_index[1].astype(jnp.int32)
    ppw = PW - E // NW
    pads = (jnp.arange(NW * ppw, dtype=jnp.int32) % (NP - N)) + PAD_ROW
    pads = pads.reshape(NW, ppw)
    srcp = jnp.concatenate([src.reshape(NW, E // NW), pads], axis=1).reshape(-1)
    dstp = jnp.concatenate([dst.reshape(NW, E // NW), pads], axis=1).reshape(-1)
    xpad = jnp.pad(x, ((0, NP - N), (0, 0)))
    zero_rows = jnp.zeros((RPT, HID), jnp.float32)

    degp = _deg_kernel(dstp)
    degp_t = degp.T
    dis, y1 = _prep(degp_t, xpad, W1)

    p1 = _agg_kernel(y1, srcp, dstp, zero_rows)
    y2 = _mid(p1, y1, dis, b1.reshape(1, HID), W2)
    p2 = _agg_kernel(y2, srcp, dstp, zero_rows)
    y3 = _mid(p2, y2, dis, b2.reshape(1, HID), W3)
    p3 = _agg_kernel(y3, srcp, dstp, zero_rows)
    h3 = _combine(p3, y3, dis, b3.reshape(1, HID))

    return _fc(h3[:N], Wfc, bfc.reshape(1, OUT_DIM))


# speedup vs baseline: 3.8598x; 1.7686x over previous
import functools

import jax
import jax.numpy as jnp
from jax import lax
from jax.experimental import pallas as pl
from jax.experimental.pallas import tpu as pltpu
from jax.experimental.pallas import tpu_sc as plsc

N = 10000
E = 640000
IN_DIM = 16
HID = 128
OUT_DIM = 10000

NP = 10112
PAD_ROW = N
NW = 32
K = 128
CPW = 160
PW = CPW * K
EPAD = NW * PW
IB = 16
RPT = NP // 16

_mesh = plsc.VectorSubcoreMesh(core_axis_name="c", subcore_axis_name="s")


@functools.partial(
    pl.kernel,
    out_type=jax.ShapeDtypeStruct((NW, NP), jnp.float32),
    mesh=_mesh,
    scratch_types=[
        pltpu.VMEM((K,), jnp.int32),
        pltpu.VMEM((NP,), jnp.float32),
    ],
    compiler_params=pltpu.CompilerParams(needs_layout_passes=False),
)
def _deg_kernel(dst_hbm, out_hbm, idx_d, acc):
    c = lax.axis_index("c")
    s = lax.axis_index("s")
    wid = c * 16 + s
    zeros16 = jnp.zeros((16,), jnp.float32)
    ones16 = jnp.ones((16,), jnp.float32)

    def zbody(i, carry):
        acc[pl.ds(i * 16, 16)] = zeros16
        return carry

    lax.fori_loop(0, NP // 16, zbody, 0)

    base0 = pl.multiple_of(wid * PW, 8)

    def body(i, carry):
        base = pl.multiple_of(base0 + i * K, 8)
        pltpu.sync_copy(dst_hbm.at[pl.ds(base, K)], idx_d)
        for j in range(K // 16):
            idx16 = idx_d[pl.ds(j * 16, 16)]
            plsc.addupdate_scatter(acc, [idx16], ones16)
        return carry

    lax.fori_loop(0, CPW, body, 0)
    pltpu.sync_copy(acc, out_hbm.at[wid])


@functools.partial(
    pl.kernel,
    out_type=jax.ShapeDtypeStruct((2, NP, HID), jnp.float32),
    mesh=_mesh,
    scratch_types=[
        pltpu.VMEM((IB, K), jnp.int32),
        pltpu.VMEM((IB, K), jnp.int32),
        pltpu.VMEM((K, HID), jnp.float32),
        pltpu.VMEM((K, HID), jnp.float32),
        pltpu.VMEM_SHARED((NP, HID), jnp.float32),
        pltpu.SemaphoreType.DMA,
        pltpu.SemaphoreType.DMA,
    ],
)
def _agg_kernel(y_hbm, src_hbm, dst_hbm, zero_hbm, out_hbm,
                src_blk, dst_blk, rows0, rows1, acc, sem0, sem1):
    c = lax.axis_index("c")
    s = lax.axis_index("s")
    wid = c * 16 + s

    row0 = pl.multiple_of(s * RPT, 8)
    pltpu.sync_copy(zero_hbm, acc.at[pl.ds(row0, RPT)])

    cbase = pl.multiple_of(wid * CPW, 8)
    plsc.subcore_barrier()

    rows = (rows0, rows1)
    sems = (sem0, sem1)

    def gather_start(t, b):
        pltpu.async_copy(y_hbm.at[src_blk.at[t]], rows[b], sems[b])

    def gather_wait(t, b):
        pltpu.make_async_copy(y_hbm.at[src_blk.at[t]], rows[b], sems[b]).wait()

    def scat(t, b):
        pltpu.sync_copy(rows[b], acc.at[dst_blk.at[t]], add=True)

    def body(j, carry):
        blk = pl.multiple_of(cbase + j * IB, 8)
        pltpu.sync_copy(src_hbm.at[pl.ds(blk, IB)], src_blk)
        pltpu.sync_copy(dst_hbm.at[pl.ds(blk, IB)], dst_blk)
        gather_start(0, 0)
        for t in range(IB - 1):
            gather_start(t + 1, (t + 1) % 2)
            gather_wait(t, t % 2)
            scat(t, t % 2)
        gather_wait(IB - 1, (IB - 1) % 2)
        scat(IB - 1, (IB - 1) % 2)
        return carry

    lax.fori_loop(0, CPW // IB, body, 0)

    plsc.subcore_barrier()
    pltpu.sync_copy(acc.at[pl.ds(row0, RPT)],
                    out_hbm.at[c, pl.ds(row0, RPT)])


def _prep_body(degp_ref, x_ref, w1_ref, dis_ref, y1_ref):
    deg = jnp.sum(degp_ref[...], axis=1, keepdims=True) + 1.0
    dis = lax.rsqrt(deg)
    dis_ref[...] = dis
    xw = jnp.dot(x_ref[...], w1_ref[...], preferred_element_type=jnp.float32)
    y1_ref[...] = dis * xw


def _prep(degp_t, xpad, w1):
    return pl.pallas_call(
        _prep_body,
        out_shape=(
            jax.ShapeDtypeStruct((NP, 1), jnp.float32),
            jax.ShapeDtypeStruct((NP, HID), jnp.float32),
        ),
    )(degp_t, xpad, w1)


_MR = 2528


def _mid_body(aggp_ref, y_ref, dis_ref, b_ref, w_ref, out_ref):
    dis = dis_ref[...]
    tot = aggp_ref[0] + aggp_ref[1] + y_ref[...]
    h = jnp.maximum(dis * tot + b_ref[...], 0.0)
    out_ref[...] = dis * jnp.dot(h, w_ref[...],
                                 preferred_element_type=jnp.float32)


def _mid(aggp, y, dis, b, w):
    return pl.pallas_call(
        _mid_body,
        grid=(NP // _MR,),
        in_specs=[
            pl.BlockSpec((2, _MR, HID), lambda i: (0, i, 0)),
            pl.BlockSpec((_MR, HID), lambda i: (i, 0)),
            pl.BlockSpec((_MR, 1), lambda i: (i, 0)),
            pl.BlockSpec((1, HID), lambda i: (0, 0)),
            pl.BlockSpec((HID, HID), lambda i: (0, 0)),
        ],
        out_specs=pl.BlockSpec((_MR, HID), lambda i: (i, 0)),
        out_shape=jax.ShapeDtypeStruct((NP, HID), jnp.float32),
    )(aggp, y, dis, b, w)


def _combine_body(aggp_ref, y_ref, dis_ref, b_ref, out_ref):
    dis = dis_ref[...]
    tot = aggp_ref[0] + aggp_ref[1] + y_ref[...]
    out_ref[...] = jnp.maximum(dis * tot + b_ref[...], 0.0)


def _combine(aggp, y, dis, b):
    return pl.pallas_call(
        _combine_body,
        grid=(NP // _MR,),
        in_specs=[
            pl.BlockSpec((2, _MR, HID), lambda i: (0, i, 0)),
            pl.BlockSpec((_MR, HID), lambda i: (i, 0)),
            pl.BlockSpec((_MR, 1), lambda i: (i, 0)),
            pl.BlockSpec((1, HID), lambda i: (0, 0)),
        ],
        out_specs=pl.BlockSpec((_MR, HID), lambda i: (i, 0)),
        out_shape=jax.ShapeDtypeStruct((NP, HID), jnp.float32),
    )(aggp, y, dis, b)


_FR = 2000
_FC = 2048


def _fc_body(h_ref, w_ref, b_ref, out_ref):
    out_ref[...] = jnp.dot(h_ref[...], w_ref[...],
                           preferred_element_type=jnp.float32) + b_ref[...]


def _fc(h, wfc, bfc):
    return pl.pallas_call(
        _fc_body,
        grid=(pl.cdiv(OUT_DIM, _FC), N // _FR),
        in_specs=[
            pl.BlockSpec((_FR, HID), lambda j, i: (i, 0)),
            pl.BlockSpec((HID, _FC), lambda j, i: (0, j)),
            pl.BlockSpec((1, _FC), lambda j, i: (0, j)),
        ],
        out_specs=pl.BlockSpec((_FR, _FC), lambda j, i: (i, j)),
        out_shape=jax.ShapeDtypeStruct((N, OUT_DIM), jnp.float32),
    )(h, wfc, bfc)


def kernel(x, edge_index, W1, b1, W2, b2, W3, b3, Wfc, bfc):
    src = edge_index[0].astype(jnp.int32)
    dst = edge_index[1].astype(jnp.int32)
    ppw = PW - E // NW
    pads = (jnp.arange(NW * ppw, dtype=jnp.int32) % (NP - N)) + PAD_ROW
    pads = pads.reshape(NW, ppw)
    srcp = jnp.concatenate([src.reshape(NW, E // NW), pads], axis=1).reshape(-1)
    dstp = jnp.concatenate([dst.reshape(NW, E // NW), pads], axis=1).reshape(-1)
    xpad = jnp.pad(x, ((0, NP - N), (0, 0)))
    zero_rows = jnp.zeros((RPT, HID), jnp.float32)

    degp = _deg_kernel(dstp)
    degp_t = degp.T
    dis, y1 = _prep(degp_t, xpad, W1)

    src2 = srcp.reshape(NW * CPW, K)
    dst2 = dstp.reshape(NW * CPW, K)
    p1 = _agg_kernel(y1, src2, dst2, zero_rows)
    y2 = _mid(p1, y1, dis, b1.reshape(1, HID), W2)
    p2 = _agg_kernel(y2, src2, dst2, zero_rows)
    y3 = _mid(p2, y2, dis, b2.reshape(1, HID), W3)
    p3 = _agg_kernel(y3, src2, dst2, zero_rows)
    h3 = _combine(p3, y3, dis, b3.reshape(1, HID))

    return _fc(h3[:N], Wfc, bfc.reshape(1, OUT_DIM))


# speedup vs baseline: 4.2584x; 1.1033x over previous
import functools

import jax
import jax.numpy as jnp
from jax import lax
from jax.experimental import pallas as pl
from jax.experimental.pallas import tpu as pltpu
from jax.experimental.pallas import tpu_sc as plsc

N = 10000
E = 640000
IN_DIM = 16
HID = 128
OUT_DIM = 10000

NP = 10112
PAD_ROW = N
NW = 32
K = 128
CPW = 160
PW = CPW * K
EPAD = NW * PW
IB = 16
RPT = NP // 16

_mesh = plsc.VectorSubcoreMesh(core_axis_name="c", subcore_axis_name="s")


@functools.partial(
    pl.kernel,
    out_type=jax.ShapeDtypeStruct((NW, NP), jnp.float32),
    mesh=_mesh,
    scratch_types=[
        pltpu.VMEM((IB, K), jnp.int32),
        pltpu.VMEM((NP,), jnp.float32),
    ],
    compiler_params=pltpu.CompilerParams(needs_layout_passes=False),
)
def _deg_kernel(dst_hbm, out_hbm, idx_d, acc):
    c = lax.axis_index("c")
    s = lax.axis_index("s")
    wid = c * 16 + s
    zeros16 = jnp.zeros((16,), jnp.float32)
    ones16 = jnp.ones((16,), jnp.float32)

    def zbody(i, carry):
        acc[pl.ds(i * 16, 16)] = zeros16
        return carry

    lax.fori_loop(0, NP // 16, zbody, 0)

    cbase = pl.multiple_of(wid * CPW, 8)

    def body(j, carry):
        blk = pl.multiple_of(cbase + j * IB, 8)
        pltpu.sync_copy(dst_hbm.at[pl.ds(blk, IB)], idx_d)
        for t in range(IB):
            for h in range(K // 16):
                idx16 = idx_d[t, pl.ds(h * 16, 16)]
                plsc.addupdate_scatter(acc, [idx16], ones16)
        return carry

    lax.fori_loop(0, CPW // IB, body, 0)
    pltpu.sync_copy(acc, out_hbm.at[wid])


@functools.partial(
    pl.kernel,
    out_type=jax.ShapeDtypeStruct((2, NP, HID), jnp.float32),
    mesh=_mesh,
    scratch_types=[
        pltpu.VMEM((IB, K), jnp.int32),
        pltpu.VMEM((IB, K), jnp.int32),
        pltpu.VMEM((IB, K), jnp.int32),
        pltpu.VMEM((IB, K), jnp.int32),
        pltpu.VMEM((K, HID), jnp.float32),
        pltpu.VMEM((K, HID), jnp.float32),
        pltpu.VMEM_SHARED((NP, HID), jnp.float32),
        pltpu.SemaphoreType.DMA,
        pltpu.SemaphoreType.DMA,
        pltpu.SemaphoreType.DMA,
        pltpu.SemaphoreType.DMA,
        pltpu.SemaphoreType.DMA,
    ],
)
def _agg_kernel(y_hbm, src_hbm, dst_hbm, zero_hbm, out_hbm,
                idxa_s, idxa_d, idxb_s, idxb_d, rows0, rows1, acc,
                gsem0, gsem1, ssem0, ssem1, isem):
    c = lax.axis_index("c")
    s = lax.axis_index("s")
    wid = c * 16 + s

    row0 = pl.multiple_of(s * RPT, 8)
    pltpu.sync_copy(zero_hbm, acc.at[pl.ds(row0, RPT)])

    cbase = pl.multiple_of(wid * CPW, 8)
    plsc.subcore_barrier()

    rows = (rows0, rows1)
    gsems = (gsem0, gsem1)
    ssems = (ssem0, ssem1)

    def run_block(sb, db):
        def g_start(t):
            pltpu.async_copy(y_hbm.at[sb.at[t]], rows[t % 2], gsems[t % 2])

        def g_wait(t):
            pltpu.make_async_copy(y_hbm.at[pl.ds(0, K)], rows[t % 2],
                                  gsems[t % 2]).wait()

        def s_start(t):
            pltpu.async_copy(rows[t % 2], acc.at[db.at[t]], ssems[t % 2],
                             add=True)

        def s_wait(t):
            pltpu.make_async_copy(rows[t % 2], acc.at[pl.ds(0, K)],
                                  ssems[t % 2]).wait()

        g_start(0)
        for t in range(IB):
            if t + 1 < IB:
                if t >= 1:
                    s_wait(t - 1)
                g_start(t + 1)
            g_wait(t)
            s_start(t)
        s_wait(IB - 2)
        s_wait(IB - 1)

    def idx_fetch(j, sref, dref):
        blk = pl.multiple_of(cbase + j * IB, 8)
        pltpu.async_copy(src_hbm.at[pl.ds(blk, IB)], sref, isem)
        pltpu.async_copy(dst_hbm.at[pl.ds(blk, IB)], dref, isem)

    def idx_wait(sref, dref):
        pltpu.make_async_copy(src_hbm.at[pl.ds(0, IB)], sref, isem).wait()
        pltpu.make_async_copy(dst_hbm.at[pl.ds(0, IB)], dref, isem).wait()

    blk0 = pl.multiple_of(cbase, 8)
    pltpu.sync_copy(src_hbm.at[pl.ds(blk0, IB)], idxa_s)
    pltpu.sync_copy(dst_hbm.at[pl.ds(blk0, IB)], idxa_d)

    def body(jj, carry):
        j0 = jj * 2
        idx_fetch(j0 + 1, idxb_s, idxb_d)
        run_block(idxa_s, idxa_d)
        idx_wait(idxb_s, idxb_d)
        idx_fetch(j0 + 2, idxa_s, idxa_d)
        run_block(idxb_s, idxb_d)
        idx_wait(idxa_s, idxa_d)
        return carry

    lax.fori_loop(0, CPW // IB // 2, body, 0)

    plsc.subcore_barrier()
    pltpu.sync_copy(acc.at[pl.ds(row0, RPT)],
                    out_hbm.at[c, pl.ds(row0, RPT)])


def _prep_body(degp_ref, x_ref, w1_ref, dis_ref, y1_ref):
    deg = jnp.sum(degp_ref[...], axis=1, keepdims=True) + 1.0
    dis = lax.rsqrt(deg)
    dis_ref[...] = dis
    xw = jnp.dot(x_ref[...], w1_ref[...], preferred_element_type=jnp.float32)
    y1_ref[...] = dis * xw


def _prep(degp_t, xpad, w1):
    return pl.pallas_call(
        _prep_body,
        out_shape=(
            jax.ShapeDtypeStruct((NP, 1), jnp.float32),
            jax.ShapeDtypeStruct((NP, HID), jnp.float32),
        ),
    )(degp_t, xpad, w1)


_MR = 2528


def _mid_body(aggp_ref, y_ref, dis_ref, b_ref, w_ref, out_ref):
    dis = dis_ref[...]
    tot = aggp_ref[0] + aggp_ref[1] + y_ref[...]
    h = jnp.maximum(dis * tot + b_ref[...], 0.0)
    out_ref[...] = dis * jnp.dot(h, w_ref[...],
                                 preferred_element_type=jnp.float32)


def _mid(aggp, y, dis, b, w):
    return pl.pallas_call(
        _mid_body,
        grid=(NP // _MR,),
        in_specs=[
            pl.BlockSpec((2, _MR, HID), lambda i: (0, i, 0)),
            pl.BlockSpec((_MR, HID), lambda i: (i, 0)),
            pl.BlockSpec((_MR, 1), lambda i: (i, 0)),
            pl.BlockSpec((1, HID), lambda i: (0, 0)),
            pl.BlockSpec((HID, HID), lambda i: (0, 0)),
        ],
        out_specs=pl.BlockSpec((_MR, HID), lambda i: (i, 0)),
        out_shape=jax.ShapeDtypeStruct((NP, HID), jnp.float32),
    )(aggp, y, dis, b, w)


def _combine_body(aggp_ref, y_ref, dis_ref, b_ref, out_ref):
    dis = dis_ref[...]
    tot = aggp_ref[0] + aggp_ref[1] + y_ref[...]
    out_ref[...] = jnp.maximum(dis * tot + b_ref[...], 0.0)


def _combine(aggp, y, dis, b):
    return pl.pallas_call(
        _combine_body,
        grid=(NP // _MR,),
        in_specs=[
            pl.BlockSpec((2, _MR, HID), lambda i: (0, i, 0)),
            pl.BlockSpec((_MR, HID), lambda i: (i, 0)),
            pl.BlockSpec((_MR, 1), lambda i: (i, 0)),
            pl.BlockSpec((1, HID), lambda i: (0, 0)),
        ],
        out_specs=pl.BlockSpec((_MR, HID), lambda i: (i, 0)),
        out_shape=jax.ShapeDtypeStruct((NP, HID), jnp.float32),
    )(aggp, y, dis, b)


_FR = 2000
_FC = 2048


def _fc_body(h_ref, w_ref, b_ref, out_ref):
    out_ref[...] = jnp.dot(h_ref[...], w_ref[...],
                           preferred_element_type=jnp.float32) + b_ref[...]


def _fc(h, wfc, bfc):
    return pl.pallas_call(
        _fc_body,
        grid=(pl.cdiv(OUT_DIM, _FC), N // _FR),
        in_specs=[
            pl.BlockSpec((_FR, HID), lambda j, i: (i, 0)),
            pl.BlockSpec((HID, _FC), lambda j, i: (0, j)),
            pl.BlockSpec((1, _FC), lambda j, i: (0, j)),
        ],
        out_specs=pl.BlockSpec((_FR, _FC), lambda j, i: (i, j)),
        out_shape=jax.ShapeDtypeStruct((N, OUT_DIM), jnp.float32),
    )(h, wfc, bfc)


def kernel(x, edge_index, W1, b1, W2, b2, W3, b3, Wfc, bfc):
    src = edge_index[0].astype(jnp.int32)
    dst = edge_index[1].astype(jnp.int32)
    ppw = PW - E // NW
    pads = (jnp.arange(NW * ppw, dtype=jnp.int32) % (NP - N)) + PAD_ROW
    pads = pads.reshape(NW, ppw)
    srcp = jnp.concatenate([src.reshape(NW, E // NW), pads], axis=1).reshape(-1)
    dstp = jnp.concatenate([dst.reshape(NW, E // NW), pads], axis=1).reshape(-1)
    xpad = jnp.pad(x, ((0, NP - N), (0, 0)))
    zero_rows = jnp.zeros((RPT, HID), jnp.float32)

    extra = jnp.full((IB, K), PAD_ROW, jnp.int32)
    src2 = jnp.concatenate([srcp.reshape(NW * CPW, K), extra])
    dst2 = jnp.concatenate([dstp.reshape(NW * CPW, K), extra])

    degp = _deg_kernel(dst2)
    degp_t = degp.T
    dis, y1 = _prep(degp_t, xpad, W1)
    p1 = _agg_kernel(y1, src2, dst2, zero_rows)
    y2 = _mid(p1, y1, dis, b1.reshape(1, HID), W2)
    p2 = _agg_kernel(y2, src2, dst2, zero_rows)
    y3 = _mid(p2, y2, dis, b2.reshape(1, HID), W3)
    p3 = _agg_kernel(y3, src2, dst2, zero_rows)
    h3 = _combine(p3, y3, dis, b3.reshape(1, HID))

    return _fc(h3[:N], Wfc, bfc.reshape(1, OUT_DIM))
